# Initial kernel scaffold; baseline (speedup 1.0000x reference)
import jax
import jax.numpy as jnp
from jax.experimental import pallas as pl


def kernel(node_feature, edge_index, batch, W1_l, b1, W1_r, bn1_w, bn1_b, bn1_m, bn1_v, W2_l, b2, W2_r, Wp, bp):
    raise NotImplementedError("write your pallas kernel here")


# speedup vs baseline: 7.1403x; 7.1403x over previous
import functools

import jax
import jax.numpy as jnp
from jax import lax
from jax.experimental import pallas as pl
from jax.experimental.pallas import tpu as pltpu
from jax.experimental.pallas import tpu_sc as plsc

EPS = 1e-5
NC = 2
NS = 16
NW = NC * NS
CHUNK = 128
BM = 400


def _sc_agg_rows(n_pad, h, e):
    nchunk = e // CHUNK
    rows_per_tile = n_pad // NS
    zb = 32

    mesh = plsc.VectorSubcoreMesh(core_axis_name="c", subcore_axis_name="s")

    @functools.partial(
        pl.kernel,
        out_type=(
            jax.ShapeDtypeStruct((NC, n_pad, h), jnp.float32),
            jax.ShapeDtypeStruct((NC, n_pad), jnp.float32),
        ),
        mesh=mesh,
        scratch_types=[
            pltpu.VMEM((CHUNK,), jnp.int32),
            pltpu.VMEM((CHUNK,), jnp.int32),
            pltpu.VMEM((CHUNK, h), jnp.float32),
            pltpu.VMEM((CHUNK,), jnp.float32),
            pltpu.VMEM((zb, h), jnp.float32),
            pltpu.VMEM_SHARED((n_pad, h), jnp.float32),
            pltpu.VMEM_SHARED((n_pad,), jnp.float32),
            pltpu.SemaphoreType.DMA,
        ],
    )
    def k(y_hbm, src_hbm, dst_hbm, out_sum, out_deg,
          sidx, didx, rows, ones_v, zblk, acc_sh, deg_sh, sem):
        cid = lax.axis_index("c")
        sid = lax.axis_index("s")
        wid = sid * NC + cid
        zero16 = jnp.zeros((16,), jnp.float32)
        one16 = jnp.ones((16,), jnp.float32)
        for r in range(zb):
            for j in range(h // 16):
                zblk[r, pl.ds(j * 16, 16)] = zero16
        for j in range(CHUNK // 16):
            ones_v[pl.ds(j * 16, 16)] = one16
        row0 = sid * rows_per_tile
        def zbody(t, c):
            pltpu.sync_copy(zblk, acc_sh.at[pl.ds(row0 + t * zb, zb)])
            return c
        lax.fori_loop(0, rows_per_tile // zb, zbody, 0)
        for t in range(rows_per_tile // CHUNK):
            pltpu.sync_copy(zblk.at[0, pl.ds(0, CHUNK)] if h >= CHUNK
                            else zblk.at[0],
                            deg_sh.at[pl.ds(row0 + t * CHUNK, CHUNK)])
        plsc.subcore_barrier()
        nmine = (nchunk - wid + NW - 1) // NW
        def ebody(t, c):
            base = (wid + t * NW) * CHUNK
            pltpu.sync_copy(src_hbm.at[pl.ds(base, CHUNK)], sidx)
            pltpu.sync_copy(dst_hbm.at[pl.ds(base, CHUNK)], didx)
            pltpu.async_copy(y_hbm.at[sidx], rows, sem).wait()
            pltpu.sync_copy(rows, acc_sh.at[didx], add=True)
            pltpu.sync_copy(ones_v, deg_sh.at[didx], add=True)
            return c
        lax.fori_loop(0, nmine, ebody, 0)
        plsc.subcore_barrier()
        pltpu.sync_copy(acc_sh.at[pl.ds(row0, rows_per_tile)],
                        out_sum.at[cid, pl.ds(row0, rows_per_tile)])
        pltpu.sync_copy(deg_sh.at[pl.ds(row0, rows_per_tile)],
                        out_deg.at[cid, pl.ds(row0, rows_per_tile)])

    return k


def _sc_agg_scalar(n_pad, e):
    nchunk = e // CHUNK
    rows_per_tile = n_pad // NS

    mesh = plsc.VectorSubcoreMesh(core_axis_name="c", subcore_axis_name="s")

    @functools.partial(
        pl.kernel,
        out_type=jax.ShapeDtypeStruct((NC, n_pad), jnp.float32),
        mesh=mesh,
        scratch_types=[
            pltpu.VMEM((CHUNK,), jnp.int32),
            pltpu.VMEM((CHUNK,), jnp.int32),
            pltpu.VMEM((CHUNK,), jnp.float32),
            pltpu.VMEM((CHUNK,), jnp.float32),
            pltpu.VMEM_SHARED((n_pad,), jnp.float32),
            pltpu.SemaphoreType.DMA,
        ],
    )
    def k(z_hbm, src_hbm, dst_hbm, out_sum, sidx, didx, vals, zrow, acc_sh, sem):
        cid = lax.axis_index("c")
        sid = lax.axis_index("s")
        wid = sid * NC + cid
        zero16 = jnp.zeros((16,), jnp.float32)
        for j in range(CHUNK // 16):
            zrow[pl.ds(j * 16, 16)] = zero16
        row0 = sid * rows_per_tile
        for t in range(rows_per_tile // CHUNK):
            pltpu.sync_copy(zrow, acc_sh.at[pl.ds(row0 + t * CHUNK, CHUNK)])
        plsc.subcore_barrier()
        nmine = (nchunk - wid + NW - 1) // NW
        def ebody(t, c):
            base = (wid + t * NW) * CHUNK
            pltpu.sync_copy(src_hbm.at[pl.ds(base, CHUNK)], sidx)
            pltpu.sync_copy(dst_hbm.at[pl.ds(base, CHUNK)], didx)
            pltpu.async_copy(z_hbm.at[sidx], vals, sem).wait()
            pltpu.sync_copy(vals, acc_sh.at[didx], add=True)
            return c
        lax.fori_loop(0, nmine, ebody, 0)
        plsc.subcore_barrier()
        pltpu.sync_copy(acc_sh.at[pl.ds(row0, rows_per_tile)],
                        out_sum.at[cid, pl.ds(row0, rows_per_tile)])

    return k


def _mm1_body(x_ref, wl_ref, wr_ref, yl_ref, yr_ref):
    x = x_ref[...]
    dn = (((1,), (1,)), ((), ()))
    yl_ref[...] = lax.dot_general(x, wl_ref[...], dn,
                                  preferred_element_type=jnp.float32)
    yr_ref[...] = lax.dot_general(x, wr_ref[...], dn,
                                  preferred_element_type=jnp.float32)


def _k2_body(ps0_ref, ps1_ref, yr_ref, pd0_ref, pd1_ref, b1_ref,
             bnw_ref, bnb_ref, bnm_ref, bnv_ref,
             w2l_ref, w2r_ref, wp_ref, b2_ref, bp_ref,
             zl_ref, zr_ref):
    deg = jnp.maximum(pd0_ref[...] + pd1_ref[...], 1.0)
    agg = (ps0_ref[...] + ps1_ref[...]) / deg
    c = agg + b1_ref[...] + yr_ref[...]
    scale = bnw_ref[...] * lax.rsqrt(bnv_ref[...] + EPS)
    h1 = jnp.maximum(scale * (c - bnm_ref[...]) + bnb_ref[...], 0.0)
    dn_c0 = (((0,), (1,)), ((), ()))
    v_l = lax.dot_general(w2l_ref[...], wp_ref[...], dn_c0,
                          preferred_element_type=jnp.float32)
    v_r = lax.dot_general(w2r_ref[...], wp_ref[...], dn_c0,
                          preferred_element_type=jnp.float32)
    dn_r = (((1,), (0,)), ((), ()))
    zl_ref[...] = lax.dot_general(h1, v_l, dn_r,
                                  preferred_element_type=jnp.float32)
    cst = jnp.sum(b2_ref[...] * wp_ref[...]) + bp_ref[0, 0]
    zr_ref[...] = lax.dot_general(h1, v_r, dn_r,
                                  preferred_element_type=jnp.float32) + cst


def _k3_body(p0_ref, p1_ref, pd0_ref, pd1_ref, zr_ref, out_ref):
    deg = jnp.maximum(pd0_ref[...] + pd1_ref[...], 1.0)
    out_ref[...] = (p0_ref[...] + p1_ref[...]) / deg + zr_ref[...]


def kernel(node_feature, edge_index, batch, W1_l, b1, W1_r,
           bn1_w, bn1_b, bn1_m, bn1_v, W2_l, b2, W2_r, Wp, bp):
    n, d_in = node_feature.shape
    h = W1_l.shape[0]
    e = edge_index.shape[1]
    n_pad = ((n + NS * CHUNK - 1) // (NS * CHUNK)) * (NS * CHUNK)
    src = edge_index[0]
    dst = edge_index[1]

    f32 = jnp.float32
    nb = n // BM

    y_l, y_r = pl.pallas_call(
        _mm1_body,
        grid=(nb,),
        in_specs=[
            pl.BlockSpec((BM, d_in), lambda i: (i, 0)),
            pl.BlockSpec((h, d_in), lambda i: (0, 0)),
            pl.BlockSpec((h, d_in), lambda i: (0, 0)),
        ],
        out_specs=[
            pl.BlockSpec((BM, h), lambda i: (i, 0)),
            pl.BlockSpec((BM, h), lambda i: (i, 0)),
        ],
        out_shape=[
            jax.ShapeDtypeStruct((n, h), f32),
            jax.ShapeDtypeStruct((n, h), f32),
        ],
    )(node_feature, W1_l, W1_r)

    psum, pdeg = _sc_agg_rows(n_pad, h, e)(y_l, src, dst)
    ps0, ps1 = psum[0, :n], psum[1, :n]
    pd0 = pdeg[0, :n].reshape(n, 1)
    pd1 = pdeg[1, :n].reshape(n, 1)

    full = lambda r, c: pl.BlockSpec((r, c), lambda i: (0, 0))
    zl, zr = pl.pallas_call(
        _k2_body,
        grid=(nb,),
        in_specs=[
            pl.BlockSpec((BM, h), lambda i: (i, 0)),
            pl.BlockSpec((BM, h), lambda i: (i, 0)),
            pl.BlockSpec((BM, h), lambda i: (i, 0)),
            pl.BlockSpec((BM, 1), lambda i: (i, 0)),
            pl.BlockSpec((BM, 1), lambda i: (i, 0)),
            full(1, h), full(1, h), full(1, h), full(1, h), full(1, h),
            full(h, h), full(h, h), full(1, h), full(1, h), full(1, 1),
        ],
        out_specs=[
            pl.BlockSpec((BM, 1), lambda i: (i, 0)),
            pl.BlockSpec((BM, 1), lambda i: (i, 0)),
        ],
        out_shape=[
            jax.ShapeDtypeStruct((n, 1), f32),
            jax.ShapeDtypeStruct((n, 1), f32),
        ],
    )(ps0, ps1, y_r, pd0, pd1,
      b1.reshape(1, h), bn1_w.reshape(1, h), bn1_b.reshape(1, h),
      bn1_m.reshape(1, h), bn1_v.reshape(1, h),
      W2_l, W2_r, Wp, b2.reshape(1, h), bp.reshape(1, 1))

    p2 = _sc_agg_scalar(n_pad, e)(zl.reshape(n), src, dst)
    p20 = p2[0, :n].reshape(n, 1)
    p21 = p2[1, :n].reshape(n, 1)

    out = pl.pallas_call(
        _k3_body,
        grid=(nb,),
        in_specs=[pl.BlockSpec((BM, 1), lambda i: (i, 0))] * 5,
        out_specs=pl.BlockSpec((BM, 1), lambda i: (i, 0)),
        out_shape=jax.ShapeDtypeStruct((n, 1), f32),
    )(p20, p21, pd0, pd1, zr)
    return out
